# baseline (speedup 1.0000x reference)
import functools

import jax
import jax.numpy as jnp
from jax import lax
from jax.experimental import pallas as pl
from jax.experimental.pallas import tpu as pltpu
from jax.experimental.pallas import tpu_sc as plsc

N = 10000
E = 320000
IN_CH = 128
HID = 128
HID2 = 64
NUM_CLASSES = 2
EPS = 1e-5

NC = 2
NS = 16
NW = NC * NS
CH = 80
EPAD = E
NT = N
NCH_ES = EPAD // NW // CH
NCH_CS = EPAD // NS // CH
RPS = 624
TAIL = N - RPS * NS
TAIL_OFF = RPS * NS
NP = 10240
K = 4

_SC_PARAMS = pltpu.CompilerParams(use_tc_tiling_on_sc=False,
                                  needs_layout_passes=False)


def _seg_copy(sid, src_ref, dst_ref):
  rs = sid * RPS
  pltpu.sync_copy(src_ref.at[pl.ds(rs, RPS)], dst_ref.at[pl.ds(rs, RPS)])

  @pl.when(sid == NS - 1)
  def _():
    pltpu.sync_copy(src_ref.at[pl.ds(TAIL_OFF, TAIL)],
                    dst_ref.at[pl.ds(TAIL_OFF, TAIL)])


def _edge_loop(table, src_v, dst_v, bufs, gsems, ssems, acc, nch):
  def g_start(j, b):
    pltpu.async_copy(table.at[src_v.at[j]], bufs[b], gsems[b])

  def g_wait(b):
    pltpu.make_async_copy(table.at[src_v.at[0]], bufs[b], gsems[b]).wait()

  def s_start(j, b):
    pltpu.async_copy(bufs[b], acc.at[dst_v.at[j]], ssems[b], add=True)

  def s_wait(b):
    pltpu.make_async_copy(bufs[b], acc.at[dst_v.at[0]], ssems[b]).wait()

  for b in range(K):
    g_start(b, b)

  def body(i, carry):
    base = i * K
    for b in range(K):
      g_wait(b)
      s_start(base + b, b)
    for b in range(K):
      @pl.when(base + K + b < nch)
      def _(b=b):
        s_wait(b)
        g_start(base + K + b, b)
    return carry

  nfull = nch // K
  lax.fori_loop(0, nfull, body, 0)
  r = nch - nfull * K
  for b in range(r):
    g_wait(b)
    s_start(nfull * K + b, b)
  for b in range(K):
    s_wait(b)


def _make_sc_agg_edgesplit(D):
  mesh = plsc.VectorSubcoreMesh(core_axis_name="c", subcore_axis_name="s")

  @functools.partial(
      pl.kernel,
      out_type=jax.ShapeDtypeStruct((NC, N, D), jnp.float32),
      mesh=mesh,
      compiler_params=_SC_PARAMS,
      scratch_types=[
          pltpu.VMEM((NCH_ES, CH), jnp.int32),
          pltpu.VMEM((NCH_ES, CH), jnp.int32),
          [pltpu.VMEM((CH, D), jnp.float32)] * K,
          pltpu.VMEM_SHARED((NT, D), jnp.float32),
          [pltpu.SemaphoreType.DMA] * K,
          [pltpu.SemaphoreType.DMA] * K,
      ],
  )
  def agg(hs_hbm, edges_hbm, zeros_hbm, out_hbm, src_v, dst_v, bufs, acc,
          gsems, ssems):
    cid = lax.axis_index("c")
    sid = lax.axis_index("s")
    wid = sid * NC + cid
    pltpu.sync_copy(edges_hbm.at[0].at[wid], src_v)
    pltpu.sync_copy(edges_hbm.at[1].at[wid], dst_v)

    @pl.when(cid == 0)
    def _():
      _seg_copy(sid, hs_hbm, acc)

    @pl.when(cid != 0)
    def _():
      _seg_copy(sid, zeros_hbm, acc)

    plsc.subcore_barrier()
    _edge_loop(hs_hbm, src_v, dst_v, bufs, gsems, ssems, acc, NCH_ES)
    plsc.subcore_barrier()
    _seg_copy(sid, acc, out_hbm.at[cid])

  return agg


def _make_sc_agg_colsplit(D2):
  mesh = plsc.VectorSubcoreMesh(core_axis_name="c", subcore_axis_name="s")

  @functools.partial(
      pl.kernel,
      out_type=jax.ShapeDtypeStruct((N, 2 * D2), jnp.float32),
      mesh=mesh,
      compiler_params=_SC_PARAMS,
      scratch_types=[
          pltpu.VMEM((NCH_CS, CH), jnp.int32),
          pltpu.VMEM((NCH_CS, CH), jnp.int32),
          [pltpu.VMEM((CH, D2), jnp.float32)] * K,
          pltpu.VMEM_SHARED((NT, D2), jnp.float32),
          [pltpu.SemaphoreType.DMA] * K,
          [pltpu.SemaphoreType.DMA] * K,
      ],
  )
  def agg(hs_hbm, edges_hbm, out_hbm, src_v, dst_v, bufs, acc, gsems, ssems):
    cid = lax.axis_index("c")
    sid = lax.axis_index("s")
    pltpu.sync_copy(edges_hbm.at[0].at[sid], src_v)
    pltpu.sync_copy(edges_hbm.at[1].at[sid], dst_v)
    table = hs_hbm.at[cid]
    _seg_copy(sid, table, acc)
    plsc.subcore_barrier()
    _edge_loop(table, src_v, dst_v, bufs, gsems, ssems, acc, NCH_CS)
    plsc.subcore_barrier()
    rs = sid * RPS
    pltpu.sync_copy(acc.at[pl.ds(rs, RPS)],
                    out_hbm.at[pl.ds(rs, RPS), pl.ds(cid * D2, D2)])

    @pl.when(sid == NS - 1)
    def _():
      pltpu.sync_copy(acc.at[pl.ds(TAIL_OFF, TAIL)],
                      out_hbm.at[pl.ds(TAIL_OFF, TAIL), pl.ds(cid * D2, D2)])

  return agg


def _make_sc_deg():
  mesh = plsc.VectorSubcoreMesh(core_axis_name="c", subcore_axis_name="s")
  NG = RPS // 16

  @functools.partial(
      pl.kernel,
      out_type=jax.ShapeDtypeStruct((NC, NP), jnp.float32),
      mesh=mesh,
      compiler_params=_SC_PARAMS,
      scratch_types=[
          pltpu.VMEM((NCH_ES, CH), jnp.int32),
          pltpu.VMEM((CH, 16), jnp.float32),
          pltpu.VMEM((RPS + TAIL, 16), jnp.float32),
          pltpu.VMEM((RPS + TAIL,), jnp.float32),
          pltpu.VMEM_SHARED((NT, 16), jnp.float32),
      ],
  )
  def deg(ones_hbm, edges_hbm, zeros_hbm, out_hbm, dst_v, onesv, rowbuf,
          packed, acc):
    cid = lax.axis_index("c")
    sid = lax.axis_index("s")
    wid = sid * NC + cid
    pltpu.sync_copy(edges_hbm.at[1].at[wid], dst_v)
    pltpu.sync_copy(ones_hbm.at[pl.ds(0, CH)], onesv)

    @pl.when(cid == 0)
    def _():
      _seg_copy(sid, ones_hbm, acc)

    @pl.when(cid != 0)
    def _():
      _seg_copy(sid, zeros_hbm, acc)

    plsc.subcore_barrier()

    def body(j, carry):
      pltpu.sync_copy(onesv, acc.at[dst_v.at[j]], add=True)
      return carry

    lax.fori_loop(0, NCH_ES, body, 0)
    plsc.subcore_barrier()

    rs = sid * RPS
    pltpu.sync_copy(acc.at[pl.ds(rs, RPS)], rowbuf.at[pl.ds(0, RPS)])

    @pl.when(sid == NS - 1)
    def _():
      pltpu.sync_copy(acc.at[pl.ds(TAIL_OFF, TAIL)],
                      rowbuf.at[pl.ds(RPS, TAIL)])

    zero16 = jnp.zeros((16,), jnp.int32)
    for i in range(NG):
      rows = lax.iota(jnp.int32, 16) + (16 * i)
      packed[pl.ds(16 * i, 16)] = plsc.load_gather(rowbuf, [rows, zero16])

    @pl.when(sid == NS - 1)
    def _():
      rows = lax.iota(jnp.int32, 16) + RPS
      packed[pl.ds(RPS, 16)] = plsc.load_gather(rowbuf, [rows, zero16])

    pltpu.sync_copy(packed.at[pl.ds(0, RPS)],
                    out_hbm.at[cid].at[pl.ds(rs, RPS)])

    @pl.when(sid == NS - 1)
    def _():
      pltpu.sync_copy(packed.at[pl.ds(RPS, TAIL)],
                      out_hbm.at[cid].at[pl.ds(TAIL_OFF, TAIL)])

  return deg


def _t1_body(degp_ref, x_ref, w_ref, hs_ref, dinv_ref):
  p = degp_ref[...]
  dinvr = lax.rsqrt(p[0] + p[1])
  dinv_ref[...] = dinvr
  dcol = jnp.transpose(dinvr)[0:N]
  h = jnp.dot(x_ref[...], w_ref[...], preferred_element_type=jnp.float32)
  hs = jnp.pad(h * dcol, ((0, NT - N), (0, 0)))
  hs_ref[0] = hs[:, 0:HID // 2]
  hs_ref[1] = hs[:, HID // 2:HID]


def _bn_relu_mm(z, g_ref, be_ref, w_ref, dcol):
  mean = jnp.mean(z, axis=0, keepdims=True)
  c = z - mean
  var = jnp.mean(c * c, axis=0, keepdims=True)
  xn = jnp.maximum(c * lax.rsqrt(var + EPS) * g_ref[...] + be_ref[...], 0.0)
  hs = jnp.dot(xn, w_ref[...], preferred_element_type=jnp.float32) * dcol
  return jnp.pad(hs, ((0, NT - N), (0, 0)))


def _mid1_body(agg_ref, dinv_ref, b_ref, g_ref, be_ref, w_ref, out_ref):
  dcol = jnp.transpose(dinv_ref[...])[0:N]
  z = agg_ref[...] * dcol + b_ref[...]
  out_ref[...] = _bn_relu_mm(z, g_ref, be_ref, w_ref, dcol)


def _mid2_body(aggp_ref, dinv_ref, b_ref, g_ref, be_ref, w_ref, out_ref):
  dcol = jnp.transpose(dinv_ref[...])[0:N]
  z = (aggp_ref[0] + aggp_ref[1]) * dcol + b_ref[...]
  out_ref[...] = _bn_relu_mm(z, g_ref, be_ref, w_ref, dcol)


def _fin_body(aggp_ref, dinv8_ref, b_ref, out_ref):
  p = aggp_ref[0] + aggp_ref[1]
  li = lax.broadcasted_iota(jnp.int32, (128, 8), 0)
  ki = lax.broadcasted_iota(jnp.int32, (128, 8), 1)
  s0 = (li == ki * 16).astype(jnp.float32)
  s1 = (li == ki * 16 + 1).astype(jnp.float32)
  dinv8 = dinv8_ref[...]
  bv = b_ref[...]
  hp = lax.Precision.HIGHEST
  z0 = jnp.dot(p, s0, preferred_element_type=jnp.float32,
               precision=hp) * dinv8 + bv[0:1, 0:1]
  z1 = jnp.dot(p, s1, preferred_element_type=jnp.float32,
               precision=hp) * dinv8 + bv[0:1, 1:2]
  m = jnp.maximum(z0, z1)
  lse = m + jnp.log(jnp.exp(z0 - m) + jnp.exp(z1 - m))
  o0 = z0 - lse
  o1 = z1 - lse
  ci = lax.broadcasted_iota(jnp.int32, (8, 16), 1)
  ri = lax.broadcasted_iota(jnp.int32, (8, 16), 0)
  a0 = (ci == 2 * ri).astype(jnp.float32)
  a1 = (ci == 2 * ri + 1).astype(jnp.float32)
  out_ref[...] = (
      jnp.dot(o0, a0, preferred_element_type=jnp.float32, precision=hp) +
      jnp.dot(o1, a1, preferred_element_type=jnp.float32, precision=hp))


_agg_l1 = _make_sc_agg_colsplit(HID // 2)
_agg_l2 = _make_sc_agg_edgesplit(HID2)
_agg_l3 = _make_sc_agg_edgesplit(16)
_deg = _make_sc_deg()

_t1 = pl.pallas_call(
    _t1_body,
    out_shape=[
        jax.ShapeDtypeStruct((NC, NT, HID // 2), jnp.float32),
        jax.ShapeDtypeStruct((1, NP), jnp.float32),
    ],
)

_mid1 = pl.pallas_call(
    _mid1_body, out_shape=jax.ShapeDtypeStruct((NT, HID2), jnp.float32))
_mid2 = pl.pallas_call(
    _mid2_body, out_shape=jax.ShapeDtypeStruct((NT, 16), jnp.float32))

_fin = pl.pallas_call(
    _fin_body, out_shape=jax.ShapeDtypeStruct((N // 8, 16), jnp.float32))


def kernel(x, edge_index, W1, b1, g1, be1, W2, b2, g2, be2, W3, b3):
  ei = jnp.pad(edge_index.astype(jnp.int32), ((0, 0), (0, EPAD - E)),
               constant_values=N)
  e_es = ei.reshape(2, NW, NCH_ES, CH)
  e_cs = ei.reshape(2, NS, NCH_CS, CH)

  ones_16 = jnp.ones((N, 16), jnp.float32)
  zeros_16 = jnp.zeros((N, 16), jnp.float32)
  zeros_2 = jnp.zeros((N, HID2), jnp.float32)

  degp = _deg(ones_16, e_es, zeros_16)
  hs1, dinv = _t1(degp.reshape(NC, 1, NP), x, W1)
  agg1 = _agg_l1(hs1, e_cs)
  hs2 = _mid1(agg1, dinv, b1.reshape(1, HID), g1.reshape(1, HID),
              be1.reshape(1, HID), W2)
  agg2 = _agg_l2(hs2, e_es, zeros_2)
  W3p = jnp.pad(W3, ((0, 0), (0, 16 - NUM_CLASSES)))
  hs3 = _mid2(agg2, dinv, b2.reshape(1, HID2), g2.reshape(1, HID2),
              be2.reshape(1, HID2), W3p)
  agg3 = _agg_l3(hs3, e_es, zeros_16)
  dinv8 = dinv.reshape(NP // 8, 8)[:N // 8]
  out = _fin(agg3.reshape(NC, N // 8, 128), dinv8, b3.reshape(1, 2))
  return out.reshape(N, NUM_CLASSES)


# speedup vs baseline: 1.1065x; 1.1065x over previous
import functools

import jax
import jax.numpy as jnp
from jax import lax
from jax.experimental import pallas as pl
from jax.experimental.pallas import tpu as pltpu
from jax.experimental.pallas import tpu_sc as plsc

N = 10000
E = 320000
IN_CH = 128
HID = 128
HID2 = 64
NUM_CLASSES = 2
EPS = 1e-5

NC = 2
NS = 16
NW = NC * NS
CH = 80
EPAD = E
NT = N
NCH_ES = EPAD // NW // CH
NCH_CS = EPAD // NS // CH
RPS = 624
TAIL = N - RPS * NS
TAIL_OFF = RPS * NS
NP = 10240
K = 8

_SC_PARAMS = pltpu.CompilerParams(use_tc_tiling_on_sc=False,
                                  needs_layout_passes=False)


def _seg_copy(sid, src_ref, dst_ref):
  rs = sid * RPS
  pltpu.sync_copy(src_ref.at[pl.ds(rs, RPS)], dst_ref.at[pl.ds(rs, RPS)])

  @pl.when(sid == NS - 1)
  def _():
    pltpu.sync_copy(src_ref.at[pl.ds(TAIL_OFF, TAIL)],
                    dst_ref.at[pl.ds(TAIL_OFF, TAIL)])


def _edge_loop(table, src_v, dst_v, bufs, gsems, ssems, acc, nch):
  def g_start(j, b):
    pltpu.async_copy(table.at[src_v.at[j]], bufs[b], gsems[b])

  def g_wait(b):
    pltpu.make_async_copy(table.at[src_v.at[0]], bufs[b], gsems[b]).wait()

  def s_start(j, b):
    pltpu.async_copy(bufs[b], acc.at[dst_v.at[j]], ssems[b], add=True)

  def s_wait(b):
    pltpu.make_async_copy(bufs[b], acc.at[dst_v.at[0]], ssems[b]).wait()

  for b in range(K):
    g_start(b, b)

  def body(i, carry):
    base = i * K
    for b in range(K):
      g_wait(b)
      s_start(base + b, b)
    for b in range(K):
      @pl.when(base + K + b < nch)
      def _(b=b):
        s_wait(b)
        g_start(base + K + b, b)
    return carry

  nfull = nch // K
  lax.fori_loop(0, nfull, body, 0)
  r = nch - nfull * K
  for b in range(r):
    g_wait(b)
    s_start(nfull * K + b, b)
  for b in range(K):
    s_wait(b)


def _make_sc_agg_edgesplit(D):
  mesh = plsc.VectorSubcoreMesh(core_axis_name="c", subcore_axis_name="s")

  @functools.partial(
      pl.kernel,
      out_type=jax.ShapeDtypeStruct((NC, N, D), jnp.float32),
      mesh=mesh,
      compiler_params=_SC_PARAMS,
      scratch_types=[
          pltpu.VMEM((NCH_ES, CH), jnp.int32),
          pltpu.VMEM((NCH_ES, CH), jnp.int32),
          [pltpu.VMEM((CH, D), jnp.float32)] * K,
          pltpu.VMEM_SHARED((NT, D), jnp.float32),
          [pltpu.SemaphoreType.DMA] * K,
          [pltpu.SemaphoreType.DMA] * K,
      ],
  )
  def agg(hs_hbm, edges_hbm, zeros_hbm, out_hbm, src_v, dst_v, bufs, acc,
          gsems, ssems):
    cid = lax.axis_index("c")
    sid = lax.axis_index("s")
    wid = sid * NC + cid
    pltpu.sync_copy(edges_hbm.at[0].at[wid], src_v)
    pltpu.sync_copy(edges_hbm.at[1].at[wid], dst_v)

    @pl.when(cid == 0)
    def _():
      _seg_copy(sid, hs_hbm, acc)

    @pl.when(cid != 0)
    def _():
      _seg_copy(sid, zeros_hbm, acc)

    plsc.subcore_barrier()
    _edge_loop(hs_hbm, src_v, dst_v, bufs, gsems, ssems, acc, NCH_ES)
    plsc.subcore_barrier()
    _seg_copy(sid, acc, out_hbm.at[cid])

  return agg


def _make_sc_agg_colsplit(D2):
  mesh = plsc.VectorSubcoreMesh(core_axis_name="c", subcore_axis_name="s")

  @functools.partial(
      pl.kernel,
      out_type=jax.ShapeDtypeStruct((N, 2 * D2), jnp.float32),
      mesh=mesh,
      compiler_params=_SC_PARAMS,
      scratch_types=[
          pltpu.VMEM((NCH_CS, CH), jnp.int32),
          pltpu.VMEM((NCH_CS, CH), jnp.int32),
          [pltpu.VMEM((CH, D2), jnp.float32)] * K,
          pltpu.VMEM_SHARED((NT, D2), jnp.float32),
          [pltpu.SemaphoreType.DMA] * K,
          [pltpu.SemaphoreType.DMA] * K,
      ],
  )
  def agg(hs_hbm, edges_hbm, out_hbm, src_v, dst_v, bufs, acc, gsems, ssems):
    cid = lax.axis_index("c")
    sid = lax.axis_index("s")
    pltpu.sync_copy(edges_hbm.at[0].at[sid], src_v)
    pltpu.sync_copy(edges_hbm.at[1].at[sid], dst_v)
    table = hs_hbm.at[cid]
    _seg_copy(sid, table, acc)
    plsc.subcore_barrier()
    _edge_loop(table, src_v, dst_v, bufs, gsems, ssems, acc, NCH_CS)
    plsc.subcore_barrier()
    rs = sid * RPS
    pltpu.sync_copy(acc.at[pl.ds(rs, RPS)],
                    out_hbm.at[pl.ds(rs, RPS), pl.ds(cid * D2, D2)])

    @pl.when(sid == NS - 1)
    def _():
      pltpu.sync_copy(acc.at[pl.ds(TAIL_OFF, TAIL)],
                      out_hbm.at[pl.ds(TAIL_OFF, TAIL), pl.ds(cid * D2, D2)])

  return agg


def _make_sc_deg():
  mesh = plsc.VectorSubcoreMesh(core_axis_name="c", subcore_axis_name="s")
  NG = RPS // 16

  @functools.partial(
      pl.kernel,
      out_type=jax.ShapeDtypeStruct((NC, NP), jnp.float32),
      mesh=mesh,
      compiler_params=_SC_PARAMS,
      scratch_types=[
          pltpu.VMEM((NCH_ES, CH), jnp.int32),
          pltpu.VMEM((CH, 16), jnp.float32),
          pltpu.VMEM((RPS + TAIL, 16), jnp.float32),
          pltpu.VMEM((RPS + TAIL,), jnp.float32),
          pltpu.VMEM_SHARED((NT, 16), jnp.float32),
      ],
  )
  def deg(ones_hbm, edges_hbm, zeros_hbm, out_hbm, dst_v, onesv, rowbuf,
          packed, acc):
    cid = lax.axis_index("c")
    sid = lax.axis_index("s")
    wid = sid * NC + cid
    pltpu.sync_copy(edges_hbm.at[1].at[wid], dst_v)
    pltpu.sync_copy(ones_hbm.at[pl.ds(0, CH)], onesv)

    @pl.when(cid == 0)
    def _():
      _seg_copy(sid, ones_hbm, acc)

    @pl.when(cid != 0)
    def _():
      _seg_copy(sid, zeros_hbm, acc)

    plsc.subcore_barrier()

    def body(j, carry):
      pltpu.sync_copy(onesv, acc.at[dst_v.at[j]], add=True)
      return carry

    lax.fori_loop(0, NCH_ES, body, 0)
    plsc.subcore_barrier()

    rs = sid * RPS
    pltpu.sync_copy(acc.at[pl.ds(rs, RPS)], rowbuf.at[pl.ds(0, RPS)])

    @pl.when(sid == NS - 1)
    def _():
      pltpu.sync_copy(acc.at[pl.ds(TAIL_OFF, TAIL)],
                      rowbuf.at[pl.ds(RPS, TAIL)])

    zero16 = jnp.zeros((16,), jnp.int32)
    for i in range(NG):
      rows = lax.iota(jnp.int32, 16) + (16 * i)
      packed[pl.ds(16 * i, 16)] = plsc.load_gather(rowbuf, [rows, zero16])

    @pl.when(sid == NS - 1)
    def _():
      rows = lax.iota(jnp.int32, 16) + RPS
      packed[pl.ds(RPS, 16)] = plsc.load_gather(rowbuf, [rows, zero16])

    pltpu.sync_copy(packed.at[pl.ds(0, RPS)],
                    out_hbm.at[cid].at[pl.ds(rs, RPS)])

    @pl.when(sid == NS - 1)
    def _():
      pltpu.sync_copy(packed.at[pl.ds(RPS, TAIL)],
                      out_hbm.at[cid].at[pl.ds(TAIL_OFF, TAIL)])

  return deg


def _t1_body(degp_ref, x_ref, w_ref, hs_ref, dinv_ref):
  p = degp_ref[...]
  dinvr = lax.rsqrt(p[0] + p[1])
  dinv_ref[...] = dinvr
  dcol = jnp.transpose(dinvr)[0:N]
  h = jnp.dot(x_ref[...], w_ref[...], preferred_element_type=jnp.float32)
  hs = jnp.pad(h * dcol, ((0, NT - N), (0, 0)))
  hs_ref[0] = hs[:, 0:HID // 2]
  hs_ref[1] = hs[:, HID // 2:HID]


def _bn_relu_mm(z, g_ref, be_ref, w_ref, dcol):
  mean = jnp.mean(z, axis=0, keepdims=True)
  c = z - mean
  var = jnp.mean(c * c, axis=0, keepdims=True)
  xn = jnp.maximum(c * lax.rsqrt(var + EPS) * g_ref[...] + be_ref[...], 0.0)
  hs = jnp.dot(xn, w_ref[...], preferred_element_type=jnp.float32) * dcol
  return jnp.pad(hs, ((0, NT - N), (0, 0)))


def _mid1_body(agg_ref, dinv_ref, b_ref, g_ref, be_ref, w_ref, out_ref):
  dcol = jnp.transpose(dinv_ref[...])[0:N]
  z = agg_ref[...] * dcol + b_ref[...]
  out_ref[...] = _bn_relu_mm(z, g_ref, be_ref, w_ref, dcol)


def _mid2_body(aggp_ref, dinv_ref, b_ref, g_ref, be_ref, w_ref, out_ref):
  dcol = jnp.transpose(dinv_ref[...])[0:N]
  z = (aggp_ref[0] + aggp_ref[1]) * dcol + b_ref[...]
  out_ref[...] = _bn_relu_mm(z, g_ref, be_ref, w_ref, dcol)


def _fin_body(aggp_ref, dinv8_ref, b_ref, out_ref):
  p = aggp_ref[0] + aggp_ref[1]
  li = lax.broadcasted_iota(jnp.int32, (128, 8), 0)
  ki = lax.broadcasted_iota(jnp.int32, (128, 8), 1)
  s0 = (li == ki * 16).astype(jnp.float32)
  s1 = (li == ki * 16 + 1).astype(jnp.float32)
  dinv8 = dinv8_ref[...]
  bv = b_ref[...]
  z0 = jnp.dot(p, s0, preferred_element_type=jnp.float32) * dinv8 + bv[0:1, 0:1]
  z1 = jnp.dot(p, s1, preferred_element_type=jnp.float32) * dinv8 + bv[0:1, 1:2]
  m = jnp.maximum(z0, z1)
  lse = m + jnp.log(jnp.exp(z0 - m) + jnp.exp(z1 - m))
  o0 = z0 - lse
  o1 = z1 - lse
  ci = lax.broadcasted_iota(jnp.int32, (8, 16), 1)
  ri = lax.broadcasted_iota(jnp.int32, (8, 16), 0)
  a0 = (ci == 2 * ri).astype(jnp.float32)
  a1 = (ci == 2 * ri + 1).astype(jnp.float32)
  out_ref[...] = (jnp.dot(o0, a0, preferred_element_type=jnp.float32) +
                  jnp.dot(o1, a1, preferred_element_type=jnp.float32))


_agg_l1 = _make_sc_agg_colsplit(HID // 2)
_agg_l2 = _make_sc_agg_edgesplit(HID2)
_agg_l3 = _make_sc_agg_edgesplit(16)
_deg = _make_sc_deg()

_t1 = pl.pallas_call(
    _t1_body,
    out_shape=[
        jax.ShapeDtypeStruct((NC, NT, HID // 2), jnp.float32),
        jax.ShapeDtypeStruct((1, NP), jnp.float32),
    ],
)

_mid1 = pl.pallas_call(
    _mid1_body, out_shape=jax.ShapeDtypeStruct((NT, HID2), jnp.float32))
_mid2 = pl.pallas_call(
    _mid2_body, out_shape=jax.ShapeDtypeStruct((NT, 16), jnp.float32))

_fin = pl.pallas_call(
    _fin_body, out_shape=jax.ShapeDtypeStruct((N // 8, 16), jnp.float32))


def kernel(x, edge_index, W1, b1, g1, be1, W2, b2, g2, be2, W3, b3):
  ei = jnp.pad(edge_index.astype(jnp.int32), ((0, 0), (0, EPAD - E)),
               constant_values=N)
  e_es = ei.reshape(2, NW, NCH_ES, CH)
  e_cs = ei.reshape(2, NS, NCH_CS, CH)

  ones_16 = jnp.ones((N, 16), jnp.float32)
  zeros_16 = jnp.zeros((N, 16), jnp.float32)
  zeros_2 = jnp.zeros((N, HID2), jnp.float32)

  degp = _deg(ones_16, e_es, zeros_16)
  hs1, dinv = _t1(degp.reshape(NC, 1, NP), x, W1)
  agg1 = _agg_l1(hs1, e_cs)
  hs2 = _mid1(agg1, dinv, b1.reshape(1, HID), g1.reshape(1, HID),
              be1.reshape(1, HID), W2)
  agg2 = _agg_l2(hs2, e_es, zeros_2)
  W3p = jnp.pad(W3, ((0, 0), (0, 16 - NUM_CLASSES)))
  hs3 = _mid2(agg2, dinv, b2.reshape(1, HID2), g2.reshape(1, HID2),
              be2.reshape(1, HID2), W3p)
  agg3 = _agg_l3(hs3, e_es, zeros_16)
  dinv8 = dinv.reshape(NP // 8, 8)[:N // 8]
  out = _fin(agg3.reshape(NC, N // 8, 128), dinv8, b3.reshape(1, 2))
  return out.reshape(N, NUM_CLASSES)
